# baseline (speedup 1.0000x reference)
import jax
import jax.numpy as jnp
from jax.experimental import pallas as pl
from jax.experimental.pallas import tpu as pltpu


def _body(x_ref, adj_ref, w1_ref, b1_ref, w2_ref, b2_ref,
          wmu_ref, bmu_ref, wlv_ref, blv_ref, eps_ref,
          wa1_ref, ba1_ref, wa2_ref, ba2_ref,
          ws1_ref, bs1_ref, ws2_ref, bs2_ref,
          ahat_ref, xhat_ref,
          xw1_s, hw2_s, s_s):
    i = pl.program_id(0)
    f32 = jnp.float32

    @pl.when(i < 5)
    def _p1():
        r = i * 2000
        xw1_s[pl.ds(r, 2000), :] = jnp.dot(
            x_ref[...], w1_ref[...], preferred_element_type=f32)

    @pl.when(jnp.logical_and(i >= 5, i < 55))
    def _p2():
        h = jnp.dot(adj_ref[...], xw1_s[...], preferred_element_type=f32)
        h = jax.nn.relu(h + b1_ref[...])
        hw2_s[pl.ds((i - 5) * 200, 200), :] = jnp.dot(
            h, w2_ref[...], preferred_element_type=f32)

    @pl.when(jnp.logical_and(i >= 55, i < 105))
    def _p3():
        r = (i - 55) * 200
        h = jnp.dot(adj_ref[...], hw2_s[...], preferred_element_type=f32)
        h = jax.nn.relu(h + b2_ref[...])
        mu = jnp.dot(h, wmu_ref[...], preferred_element_type=f32) + bmu_ref[...]
        lv = jnp.dot(h, wlv_ref[...], preferred_element_type=f32) + blv_ref[...]
        z = mu + eps_ref[pl.ds(r, 200), :] * jnp.exp(0.5 * lv)
        a = jax.nn.relu(jnp.dot(z, wa1_ref[...], preferred_element_type=f32)
                        + ba1_ref[...])
        xhat_ref[...] = jnp.dot(a, wa2_ref[...],
                                preferred_element_type=f32) + ba2_ref[...]
        s = jax.nn.relu(jnp.dot(z, ws1_ref[...], preferred_element_type=f32)
                        + bs1_ref[...])
        s_s[pl.ds(r, 200), :] = jnp.dot(
            s, ws2_ref[...], preferred_element_type=f32) + bs2_ref[...]

    @pl.when(i >= 105)
    def _p4():
        r = (i - 105) * 200
        logits = jax.lax.dot_general(
            s_s[pl.ds(r, 200), :], s_s[...], (((1,), (1,)), ((), ())),
            preferred_element_type=f32)
        ahat_ref[...] = jax.nn.sigmoid(logits)


def kernel(x, adj, W1, b1, W2, b2, Wmu, bmu, Wlv, blv,
           Wa1, ba1, Wa2, ba2, Ws1, bs1, Ws2, bs2):
    N, F = x.shape
    H = W1.shape[1]
    L = Wmu.shape[1]
    f32 = jnp.float32

    b1r = b1.reshape(1, H); b2r = b2.reshape(1, H)
    bmur = bmu.reshape(1, L); blvr = blv.reshape(1, L)
    ba1r = ba1.reshape(1, L); ba2r = ba2.reshape(1, F)
    bs1r = bs1.reshape(1, L); bs2r = bs2.reshape(1, L)
    eps = jax.random.normal(jax.random.key(42), (N, L), f32)

    def pin(a):
        return pl.BlockSpec(a.shape, lambda i: (0, 0))

    def adj_idx(i):
        return (jnp.where(i < 55, jnp.maximum(i - 5, 0),
                          jnp.where(i < 105, i - 55, 49)), 0)

    A_hat, x_hat = pl.pallas_call(
        _body,
        grid=(155,),
        in_specs=[
            pl.BlockSpec((2000, F), lambda i: (jnp.minimum(i, 4), 0)),
            pl.BlockSpec((200, N), adj_idx),
            pin(W1), pin(b1r), pin(W2), pin(b2r),
            pin(Wmu), pin(bmur), pin(Wlv), pin(blvr), pin(eps),
            pin(Wa1), pin(ba1r), pin(Wa2), pin(ba2r),
            pin(Ws1), pin(bs1r), pin(Ws2), pin(bs2r),
        ],
        out_specs=[
            pl.BlockSpec((200, N), lambda i: (jnp.maximum(i - 105, 0), 0)),
            pl.BlockSpec((200, F),
                         lambda i: (jnp.clip(i - 55, 0, 49), 0)),
        ],
        out_shape=[jax.ShapeDtypeStruct((N, N), f32),
                   jax.ShapeDtypeStruct((N, F), f32)],
        scratch_shapes=[pltpu.VMEM((N, H), f32),
                        pltpu.VMEM((N, H), f32),
                        pltpu.VMEM((N, L), f32)],
    )(x, adj, W1, b1r, W2, b2r, Wmu, bmur, Wlv, blvr, eps,
      Wa1, ba1r, Wa2, ba2r, Ws1, bs1r, Ws2, bs2r)

    return (A_hat, x_hat)
